# baseline (speedup 1.0000x reference)
import functools

import jax
import jax.numpy as jnp
from jax import lax
from jax.experimental import pallas as pl
from jax.experimental.pallas import tpu as pltpu
from jax.experimental.pallas import tpu_sc as plsc

N_VARS = 100000
N_CLAUSES = 100000
E = 3200000

NC = 2
NS = 16
L = 16
NW = NC * NS

WORDS = 4096
LOG2W = WORDS.bit_length() - 1
VPAD = 32 * WORDS
CPAD = 100096
CHUNK = 6400
NCHUNK = E // CHUNK
SLOTS = 16
NPAIR = SLOTS // 2
UNROLL = 8
ZUNROLL = 8
THRESH = 0.50001
RGRID = 17
RBLK = CPAD // RGRID


def _pack_body(x_ref, out_ref):
    x = x_ref[...]
    b = jnp.where(x >= THRESH, 1, 0).astype(jnp.int32)
    shifts = lax.broadcasted_iota(jnp.int32, (32, 1), 0)
    out_ref[...] = jnp.sum(b << shifts, axis=0, keepdims=True)


_pack_call = pl.pallas_call(
    _pack_body,
    out_shape=jax.ShapeDtypeStruct((1, WORDS), jnp.int32),
)


_sc_mesh = plsc.VectorSubcoreMesh(core_axis_name="c", subcore_axis_name="s")


@functools.partial(
    pl.kernel,
    out_type=jax.ShapeDtypeStruct((NW, CPAD), jnp.int32),
    mesh=_sc_mesh,
    compiler_params=pltpu.CompilerParams(needs_layout_passes=False),
    scratch_types=[
        pltpu.VMEM((WORDS,), jnp.int32),
        pltpu.VMEM((CPAD,), jnp.int32),
        pltpu.VMEM((2, CHUNK), jnp.int32),
        pltpu.VMEM((2, CHUNK), jnp.int32),
        pltpu.SemaphoreType.DMA,
        pltpu.SemaphoreType.DMA,
    ],
)
def _scatter_kernel(packed_hbm, pos_hbm, neg_hbm, out_hbm,
                    packed_v, acc_v, e0_v, e1_v, s0, s1):
    wid = lax.axis_index("s") * NC + lax.axis_index("c")
    e_b = (e0_v, e1_v)
    s_b = (s0, s1)

    def chunk_id(k):
        return wid + k * NW

    def start(adj_hbm, bi, k):
        c = chunk_id(k)

        @pl.when(c < NCHUNK)
        def _():
            pltpu.async_copy(adj_hbm.at[:, pl.ds(c * CHUNK, CHUNK)],
                             e_b[bi], s_b[bi])

    def wait(adj_hbm, bi, k):
        @pl.when(chunk_id(k) < NCHUNK)
        def _():
            pltpu.make_async_copy(adj_hbm.at[:, pl.ds(0, CHUNK)],
                                  e_b[bi], s_b[bi]).wait()

    def process(bi, is_pos, k):
        @pl.when(chunk_id(k) < NCHUNK)
        def _():
            @plsc.parallel_loop(0, CHUNK, step=L, unroll=UNROLL)
            def _(o):
                iv = e_b[bi][1, pl.ds(o, L)]
                ic = e_b[bi][0, pl.ds(o, L)]
                word = plsc.load_gather(packed_v, [iv & (WORDS - 1)])
                sh = lax.shift_right_logical(iv, LOG2W)
                b = lax.shift_right_logical(word, sh) & 1
                val = b if is_pos else 1 - b
                plsc.addupdate_scatter(acc_v, [ic], val)

    start(pos_hbm, 0, 0)
    pltpu.sync_copy(packed_hbm, packed_v)

    zeros = jnp.zeros((L,), jnp.int32)

    @plsc.parallel_loop(0, CPAD, step=L, unroll=ZUNROLL)
    def _(o):
        acc_v[pl.ds(o, L)] = zeros

    for adj_hbm, is_pos in ((pos_hbm, True), (neg_hbm, False)):
        def pair(p, _, adj_hbm=adj_hbm, is_pos=is_pos):
            k0, k1 = 2 * p, 2 * p + 1
            wait(adj_hbm, 0, k0)
            start(adj_hbm, 1, k1)
            process(0, is_pos, k0)
            wait(adj_hbm, 1, k1)
            start(adj_hbm, 0, k0 + 2)
            process(1, is_pos, k1)
            return 0

        lax.fori_loop(0, NPAIR, pair, 0)
        if is_pos:
            start(neg_hbm, 0, 0)

    pltpu.sync_copy(acc_v, out_hbm.at[wid])


def _reduce_body(x_ref, out_ref):
    j = pl.program_id(0)
    s = jnp.sum(x_ref[...], axis=0, keepdims=True)
    cid = j * RBLK + lax.broadcasted_iota(jnp.int32, (1, RBLK), 1)
    s = jnp.where(cid < N_CLAUSES, s, jnp.int32(2**31 - 1))
    m = jnp.min(s)

    @pl.when(j == 0)
    def _():
        out_ref[0, 0] = m

    @pl.when(j > 0)
    def _():
        out_ref[0, 0] = jnp.minimum(out_ref[0, 0], m)


_reduce_call = pl.pallas_call(
    _reduce_body,
    grid=(RGRID,),
    in_specs=[pl.BlockSpec((NW, RBLK), lambda j: (0, j))],
    out_specs=pl.BlockSpec(memory_space=pltpu.SMEM),
    out_shape=jax.ShapeDtypeStruct((1, 1), jnp.int32),
)


def kernel(xv, adj_pos, adj_neg):
    xvp = jnp.pad(xv, (0, VPAD - N_VARS)).reshape(32, WORDS)
    packed = _pack_call(xvp).reshape(WORDS)
    partials = _scatter_kernel(packed, adj_pos, adj_neg)
    m = _reduce_call(partials)
    return m[0, 0].astype(jnp.float32)


# speedup vs baseline: 1.0033x; 1.0033x over previous
import functools

import jax
import jax.numpy as jnp
from jax import lax
from jax.experimental import pallas as pl
from jax.experimental.pallas import tpu as pltpu
from jax.experimental.pallas import tpu_sc as plsc

N_VARS = 100000
N_CLAUSES = 100000
E = 3200000

NC = 2
NS = 16
L = 16
NW = NC * NS

WORDS = 4096
LOG2W = WORDS.bit_length() - 1
VPAD = 32 * WORDS
CPAD = 102400
CHUNK = 5120
NCHUNK = E // CHUNK
SLOTS = 20
NPAIR = SLOTS // 2
UNROLL = 8
ZUNROLL = 8
THRESH = 0.50001
RBLK = CPAD // 8


def _pack_body(x_ref, out_ref):
    x = x_ref[...]
    b = jnp.where(x >= THRESH, 1, 0).astype(jnp.int32)
    shifts = lax.broadcasted_iota(jnp.int32, (32, 1), 0)
    out_ref[...] = jnp.sum(b << shifts, axis=0, keepdims=True)


_pack_call = pl.pallas_call(
    _pack_body,
    out_shape=jax.ShapeDtypeStruct((1, WORDS), jnp.int32),
    compiler_params=pltpu.CompilerParams(allow_input_fusion=[True]),
)


_sc_mesh = plsc.VectorSubcoreMesh(core_axis_name="c", subcore_axis_name="s")


@functools.partial(
    pl.kernel,
    out_type=jax.ShapeDtypeStruct((NW, CPAD), jnp.int32),
    mesh=_sc_mesh,
    compiler_params=pltpu.CompilerParams(needs_layout_passes=False),
    scratch_types=[
        pltpu.VMEM((WORDS,), jnp.int32),
        pltpu.VMEM((CPAD,), jnp.int32),
        pltpu.VMEM((2, CHUNK), jnp.int32),
        pltpu.VMEM((2, CHUNK), jnp.int32),
        pltpu.SemaphoreType.DMA,
        pltpu.SemaphoreType.DMA,
    ],
)
def _scatter_kernel(packed_hbm, pos_hbm, neg_hbm, out_hbm,
                    packed_v, acc_v, e0_v, e1_v, s0, s1):
    wid = lax.axis_index("s") * NC + lax.axis_index("c")
    e_b = (e0_v, e1_v)
    s_b = (s0, s1)

    def chunk_id(k):
        return wid + k * NW

    def start(adj_hbm, bi, k):
        c = chunk_id(k)

        @pl.when(c < NCHUNK)
        def _():
            pltpu.async_copy(adj_hbm.at[:, pl.ds(c * CHUNK, CHUNK)],
                             e_b[bi], s_b[bi])

    def wait(adj_hbm, bi, k):
        @pl.when(chunk_id(k) < NCHUNK)
        def _():
            pltpu.make_async_copy(adj_hbm.at[:, pl.ds(0, CHUNK)],
                                  e_b[bi], s_b[bi]).wait()

    def process(bi, is_pos, k):
        @pl.when(chunk_id(k) < NCHUNK)
        def _():
            @plsc.parallel_loop(0, CHUNK, step=L, unroll=UNROLL)
            def _(o):
                iv = e_b[bi][1, pl.ds(o, L)]
                ic = e_b[bi][0, pl.ds(o, L)]
                word = plsc.load_gather(packed_v, [iv & (WORDS - 1)])
                sh = lax.shift_right_logical(iv, LOG2W)
                b = lax.shift_right_logical(word, sh) & 1
                val = b if is_pos else 1 - b
                plsc.addupdate_scatter(acc_v, [ic], val)

    start(pos_hbm, 0, 0)
    pltpu.sync_copy(packed_hbm, packed_v)

    zeros = jnp.zeros((L,), jnp.int32)

    @plsc.parallel_loop(0, CPAD, step=L, unroll=ZUNROLL)
    def _(o):
        acc_v[pl.ds(o, L)] = zeros

    for adj_hbm, is_pos in ((pos_hbm, True), (neg_hbm, False)):
        def pair(p, _, adj_hbm=adj_hbm, is_pos=is_pos):
            k0, k1 = 2 * p, 2 * p + 1
            wait(adj_hbm, 0, k0)
            start(adj_hbm, 1, k1)
            process(0, is_pos, k0)
            wait(adj_hbm, 1, k1)
            start(adj_hbm, 0, k0 + 2)
            process(1, is_pos, k1)
            return 0

        lax.fori_loop(0, NPAIR, pair, 0)
        if is_pos:
            start(neg_hbm, 0, 0)

    pltpu.sync_copy(acc_v, out_hbm.at[wid])


def _reduce_body(x_ref, out_ref):
    j = pl.program_id(0)
    s = jnp.sum(x_ref[...], axis=0, keepdims=True)
    cid = j * RBLK + lax.broadcasted_iota(jnp.int32, (1, RBLK), 1)
    s = jnp.where(cid < N_CLAUSES, s, jnp.int32(2**31 - 1))
    m = jnp.min(s)

    @pl.when(j == 0)
    def _():
        out_ref[0, 0] = m

    @pl.when(j > 0)
    def _():
        out_ref[0, 0] = jnp.minimum(out_ref[0, 0], m)


_reduce_call = pl.pallas_call(
    _reduce_body,
    grid=(CPAD // RBLK,),
    in_specs=[pl.BlockSpec((NW, RBLK), lambda j: (0, j))],
    out_specs=pl.BlockSpec(memory_space=pltpu.SMEM),
    out_shape=jax.ShapeDtypeStruct((1, 1), jnp.int32),
)


def kernel(xv, adj_pos, adj_neg):
    xvp = jnp.pad(xv, (0, VPAD - N_VARS)).reshape(32, WORDS)
    packed = _pack_call(xvp).reshape(WORDS)
    partials = _scatter_kernel(packed, adj_pos, adj_neg)
    m = _reduce_call(partials)
    return m[0, 0].astype(jnp.float32)
